# baseline (speedup 1.0000x reference)
import functools

import jax
import jax.numpy as jnp
from jax import lax
from jax.experimental import pallas as pl
from jax.experimental.pallas import tpu as pltpu


def _msg_body(ea_ref, xj_ref, w1_ref, b1_ref, w2_ref, b2_ref, out_ref, *, i_dim,
              o_dim):
    h = jnp.maximum(
        jnp.dot(ea_ref[...], w1_ref[...], preferred_element_type=jnp.float32)
        + b1_ref[...], 0.0)
    w_flat = jnp.dot(h, w2_ref[...], preferred_element_type=jnp.float32) \
        + b2_ref[...]
    xj = xj_ref[...]
    acc = jnp.zeros((xj.shape[0], o_dim), jnp.float32)
    for i in range(i_dim):
        acc = acc + xj[:, i:i + 1] * w_flat[:, i * o_dim:(i + 1) * o_dim]
    out_ref[...] = acc


def _edge_messages(ea, xj, w1, b1, w2, b2, i_dim, o_dim, block):
    e_num = ea.shape[0]
    grid = pl.cdiv(e_num, block)
    whole = lambda *_: (0, 0)
    return pl.pallas_call(
        functools.partial(_msg_body, i_dim=i_dim, o_dim=o_dim),
        grid=(grid,),
        in_specs=[
            pl.BlockSpec((block, ea.shape[1]), lambda e: (e, 0)),
            pl.BlockSpec((block, i_dim), lambda e: (e, 0)),
            pl.BlockSpec(w1.shape, whole),
            pl.BlockSpec((1, 128), whole),
            pl.BlockSpec(w2.shape, whole),
            pl.BlockSpec((1, i_dim * o_dim), whole),
        ],
        out_specs=pl.BlockSpec((block, o_dim), lambda e: (e, 0)),
        out_shape=jax.ShapeDtypeStruct((e_num, o_dim), jnp.float32),
    )(ea, xj, w1, b1.reshape(1, -1), w2, b2.reshape(1, -1))


def _update_body(agg_ref, x_ref, root_ref, bias_ref, out_ref):
    v = agg_ref[...] + jnp.dot(
        x_ref[...], root_ref[...], preferred_element_type=jnp.float32) \
        + bias_ref[...]
    out_ref[...] = jnp.where(v > 0, v, jnp.exp(v) - 1.0)


def _node_update(agg, x, root, bias, block):
    n = x.shape[0]
    grid = pl.cdiv(n, block)
    whole = lambda *_: (0, 0)
    return pl.pallas_call(
        _update_body,
        grid=(grid,),
        in_specs=[
            pl.BlockSpec((block, root.shape[1]), lambda r: (r, 0)),
            pl.BlockSpec((block, root.shape[0]), lambda r: (r, 0)),
            pl.BlockSpec(root.shape, whole),
            pl.BlockSpec((1, root.shape[1]), whole),
        ],
        out_specs=pl.BlockSpec((block, root.shape[1]), lambda r: (r, 0)),
        out_shape=jax.ShapeDtypeStruct((n, root.shape[1]), jnp.float32),
    )(agg, x, root, bias.reshape(1, -1))


def _head_body(x_ref, w1_ref, b1_ref, w2_ref, b2_ref, w3_ref, b3_ref, out_ref):
    def elu(v):
        return jnp.where(v > 0, v, jnp.exp(v) - 1.0)

    h = elu(jnp.dot(x_ref[...], w1_ref[...],
                    preferred_element_type=jnp.float32) + b1_ref[...])
    h = elu(jnp.dot(h, w2_ref[...],
                    preferred_element_type=jnp.float32) + b2_ref[...])
    out_ref[...] = jnp.dot(h, w3_ref[...],
                           preferred_element_type=jnp.float32) + b3_ref[...]


def _head(x, fc1_w, fc1_b, fc2_w, fc2_b, fc3_w, fc3_b):
    g = x.shape[0]
    return pl.pallas_call(
        _head_body,
        out_shape=jax.ShapeDtypeStruct((g, 1), jnp.float32),
    )(x, fc1_w, fc1_b.reshape(1, -1), fc2_w, fc2_b.reshape(1, -1), fc3_w,
      fc3_b.reshape(1, -1))


def _gather_rows(x, idx):
    return jnp.take(x, idx, axis=0)


def _scatter_add(msg, dst, n):
    return jax.ops.segment_sum(msg, dst, num_segments=n)


def _mean_pool(x, seg, n):
    s = jax.ops.segment_sum(x, seg, num_segments=n)
    c = jax.ops.segment_sum(jnp.ones((x.shape[0], 1), x.dtype), seg,
                            num_segments=n)
    return s / jnp.maximum(c, 1.0)


_SUB_DIMS = [(16, 32), (32, 64), (64, 64)]
_GLOB_DIMS = [(64, 64), (64, 64)]


def _nnconv_layer(x, src, dst, ea, params, i_dim, o_dim, block):
    w1, b1, w2, b2, root, bias = params
    xj = _gather_rows(x, src)
    msg = _edge_messages(ea, xj, w1, b1, w2, b2, i_dim, o_dim, block)
    agg = _scatter_add(msg, dst, x.shape[0])
    return _node_update(agg, x, root, bias, block=512)


def kernel(x, edge_index, edge_attr, node_to_subgraph, original_edge_index,
           original_edge_attr, subgraph_to_graph, sub0_nw1, sub0_nb1, sub0_nw2,
           sub0_nb2, sub0_root, sub0_bias, sub1_nw1, sub1_nb1, sub1_nw2,
           sub1_nb2, sub1_root, sub1_bias, sub2_nw1, sub2_nb1, sub2_nw2,
           sub2_nb2, sub2_root, sub2_bias, gl0_nw1, gl0_nb1, gl0_nw2, gl0_nb2,
           gl0_root, gl0_bias, gl1_nw1, gl1_nb1, gl1_nw2, gl1_nb2, gl1_root,
           gl1_bias, fc1_w, fc1_b, fc2_w, fc2_b, fc3_w, fc3_b):
    sub_params = [
        (sub0_nw1, sub0_nb1, sub0_nw2, sub0_nb2, sub0_root, sub0_bias),
        (sub1_nw1, sub1_nb1, sub1_nw2, sub1_nb2, sub1_root, sub1_bias),
        (sub2_nw1, sub2_nb1, sub2_nw2, sub2_nb2, sub2_root, sub2_bias),
    ]
    gl_params = [
        (gl0_nw1, gl0_nb1, gl0_nw2, gl0_nb2, gl0_root, gl0_bias),
        (gl1_nw1, gl1_nb1, gl1_nw2, gl1_nb2, gl1_root, gl1_bias),
    ]

    src, dst = edge_index[0], edge_index[1]
    for l, (i_dim, o_dim) in enumerate(_SUB_DIMS):
        x = _nnconv_layer(x, src, dst, edge_attr, sub_params[l], i_dim, o_dim,
                          block=256)
    x = _mean_pool(x, node_to_subgraph, 10000)

    osrc, odst = original_edge_index[0], original_edge_index[1]
    for l, (i_dim, o_dim) in enumerate(_GLOB_DIMS):
        x = _nnconv_layer(x, osrc, odst, original_edge_attr, gl_params[l],
                          i_dim, o_dim, block=256)
    x = _mean_pool(x, subgraph_to_graph, 256)

    return _head(x, fc1_w, fc1_b, fc2_w, fc2_b, fc3_w, fc3_b).reshape(-1)


# speedup vs baseline: 1.6899x; 1.6899x over previous
import functools

import jax
import jax.numpy as jnp
from jax import lax
from jax.experimental import pallas as pl
from jax.experimental.pallas import tpu as pltpu


def _msg_body(eat_ref, xjt_ref, w1t_ref, b1t_ref, w2t_ref, b2t_ref, out_ref, *,
              i_dim, o_dim):
    ht = jnp.maximum(
        jnp.dot(w1t_ref[...], eat_ref[...], preferred_element_type=jnp.float32)
        + b1t_ref[...], 0.0)
    wt = jnp.dot(w2t_ref[...], ht, preferred_element_type=jnp.float32) \
        + b2t_ref[...]
    xjt = xjt_ref[...]
    acc = jnp.zeros((o_dim, xjt.shape[1]), jnp.float32)
    for i in range(i_dim):
        acc = acc + xjt[i:i + 1, :] * wt[i * o_dim:(i + 1) * o_dim, :]
    out_ref[...] = acc


def _edge_messages(eat, xjt, w1, b1, w2, b2, i_dim, o_dim, block):
    e_num = eat.shape[1]
    grid = pl.cdiv(e_num, block)
    whole = lambda *_: (0, 0)
    return pl.pallas_call(
        functools.partial(_msg_body, i_dim=i_dim, o_dim=o_dim),
        grid=(grid,),
        in_specs=[
            pl.BlockSpec((eat.shape[0], block), lambda e: (0, e)),
            pl.BlockSpec((i_dim, block), lambda e: (0, e)),
            pl.BlockSpec((w1.shape[1], w1.shape[0]), whole),
            pl.BlockSpec((128, 1), whole),
            pl.BlockSpec((w2.shape[1], w2.shape[0]), whole),
            pl.BlockSpec((i_dim * o_dim, 1), whole),
        ],
        out_specs=pl.BlockSpec((o_dim, block), lambda e: (0, e)),
        out_shape=jax.ShapeDtypeStruct((o_dim, e_num), jnp.float32),
    )(eat, xjt, w1.T, b1.reshape(-1, 1), w2.T, b2.reshape(-1, 1))


def _update_body(agg_ref, x_ref, root_ref, bias_ref, out_ref):
    v = agg_ref[...] + jnp.dot(
        x_ref[...], root_ref[...], preferred_element_type=jnp.float32) \
        + bias_ref[...]
    out_ref[...] = jnp.where(v > 0, v, jnp.exp(v) - 1.0)


def _node_update(agg, x, root, bias, block):
    n = x.shape[0]
    grid = pl.cdiv(n, block)
    whole = lambda *_: (0, 0)
    return pl.pallas_call(
        _update_body,
        grid=(grid,),
        in_specs=[
            pl.BlockSpec((block, root.shape[1]), lambda r: (r, 0)),
            pl.BlockSpec((block, root.shape[0]), lambda r: (r, 0)),
            pl.BlockSpec(root.shape, whole),
            pl.BlockSpec((1, root.shape[1]), whole),
        ],
        out_specs=pl.BlockSpec((block, root.shape[1]), lambda r: (r, 0)),
        out_shape=jax.ShapeDtypeStruct((n, root.shape[1]), jnp.float32),
    )(agg, x, root, bias.reshape(1, -1))


def _head_body(x_ref, w1_ref, b1_ref, w2_ref, b2_ref, w3_ref, b3_ref, out_ref):
    def elu(v):
        return jnp.where(v > 0, v, jnp.exp(v) - 1.0)

    h = elu(jnp.dot(x_ref[...], w1_ref[...],
                    preferred_element_type=jnp.float32) + b1_ref[...])
    h = elu(jnp.dot(h, w2_ref[...],
                    preferred_element_type=jnp.float32) + b2_ref[...])
    out_ref[...] = jnp.dot(h, w3_ref[...],
                           preferred_element_type=jnp.float32) + b3_ref[...]


def _head(x, fc1_w, fc1_b, fc2_w, fc2_b, fc3_w, fc3_b):
    g = x.shape[0]
    return pl.pallas_call(
        _head_body,
        out_shape=jax.ShapeDtypeStruct((g, 1), jnp.float32),
    )(x, fc1_w, fc1_b.reshape(1, -1), fc2_w, fc2_b.reshape(1, -1), fc3_w,
      fc3_b.reshape(1, -1))


def _gather_rows(x, idx):
    return jnp.take(x, idx, axis=0)


def _scatter_add(msg, dst, n):
    return jax.ops.segment_sum(msg, dst, num_segments=n)


def _mean_pool(x, seg, n):
    s = jax.ops.segment_sum(x, seg, num_segments=n)
    c = jax.ops.segment_sum(jnp.ones((x.shape[0], 1), x.dtype), seg,
                            num_segments=n)
    return s / jnp.maximum(c, 1.0)


_SUB_DIMS = [(16, 32), (32, 64), (64, 64)]
_GLOB_DIMS = [(64, 64), (64, 64)]


def _nnconv_layer(x, src, dst, eat, params, i_dim, o_dim, block):
    w1, b1, w2, b2, root, bias = params
    xjt = _gather_rows(x, src).T
    msgt = _edge_messages(eat, xjt, w1, b1, w2, b2, i_dim, o_dim, block)
    agg = _scatter_add(msgt.T, dst, x.shape[0])
    return _node_update(agg, x, root, bias, block=512)


def kernel(x, edge_index, edge_attr, node_to_subgraph, original_edge_index,
           original_edge_attr, subgraph_to_graph, sub0_nw1, sub0_nb1, sub0_nw2,
           sub0_nb2, sub0_root, sub0_bias, sub1_nw1, sub1_nb1, sub1_nw2,
           sub1_nb2, sub1_root, sub1_bias, sub2_nw1, sub2_nb1, sub2_nw2,
           sub2_nb2, sub2_root, sub2_bias, gl0_nw1, gl0_nb1, gl0_nw2, gl0_nb2,
           gl0_root, gl0_bias, gl1_nw1, gl1_nb1, gl1_nw2, gl1_nb2, gl1_root,
           gl1_bias, fc1_w, fc1_b, fc2_w, fc2_b, fc3_w, fc3_b):
    sub_params = [
        (sub0_nw1, sub0_nb1, sub0_nw2, sub0_nb2, sub0_root, sub0_bias),
        (sub1_nw1, sub1_nb1, sub1_nw2, sub1_nb2, sub1_root, sub1_bias),
        (sub2_nw1, sub2_nb1, sub2_nw2, sub2_nb2, sub2_root, sub2_bias),
    ]
    gl_params = [
        (gl0_nw1, gl0_nb1, gl0_nw2, gl0_nb2, gl0_root, gl0_bias),
        (gl1_nw1, gl1_nb1, gl1_nw2, gl1_nb2, gl1_root, gl1_bias),
    ]

    eat = edge_attr.T
    src, dst = edge_index[0], edge_index[1]
    for l, (i_dim, o_dim) in enumerate(_SUB_DIMS):
        x = _nnconv_layer(x, src, dst, eat, sub_params[l], i_dim, o_dim,
                          block=256)
    x = _mean_pool(x, node_to_subgraph, 10000)

    oeat = original_edge_attr.T
    osrc, odst = original_edge_index[0], original_edge_index[1]
    for l, (i_dim, o_dim) in enumerate(_GLOB_DIMS):
        x = _nnconv_layer(x, osrc, odst, oeat, gl_params[l],
                          i_dim, o_dim, block=256)
    x = _mean_pool(x, subgraph_to_graph, 256)

    return _head(x, fc1_w, fc1_b, fc2_w, fc2_b, fc3_w, fc3_b).reshape(-1)
